# baseline (speedup 1.0000x reference)
import functools
import math

import jax
import jax.numpy as jnp
from jax.experimental import pallas as pl

_BLK = 1000


def _apply_act(h, act):
    if act == "relu":
        return jnp.maximum(h, 0.0)
    if act in ("softmax", "relu_softmax"):
        if act == "relu_softmax":
            h = jnp.maximum(h, 0.0)
        m = jnp.max(h, axis=1, keepdims=True)
        e = jnp.exp(h - m)
        return e / jnp.sum(e, axis=1, keepdims=True)
    if act == "softmax5concat":
        col = jax.lax.broadcasted_iota(jnp.int32, h.shape, 1)
        in5 = col < 5
        z = jnp.where(in5, h, -jnp.inf)
        m = jnp.max(z, axis=1, keepdims=True)
        e = jnp.where(in5, jnp.exp(z - m), 0.0)
        sm = e / jnp.sum(e, axis=1, keepdims=True)
        return jnp.where(in5, sm, h)
    return h


def _mm_body(x_ref, w_ref, b_ref, o_ref, *, act):
    h = jnp.dot(x_ref[...], w_ref[...],
                preferred_element_type=jnp.float32,
                precision=jax.lax.Precision.HIGHEST)
    o_ref[...] = _apply_act(h + b_ref[...], act)


def _mm(x, w, b, act="none"):
    n, fi = x.shape
    fo = w.shape[1]
    blk = min(_BLK, n)
    return pl.pallas_call(
        functools.partial(_mm_body, act=act),
        grid=(pl.cdiv(n, blk),),
        in_specs=[
            pl.BlockSpec((blk, fi), lambda i: (i, 0)),
            pl.BlockSpec((fi, fo), lambda i: (0, 0)),
            pl.BlockSpec((1, fo), lambda i: (0, 0)),
        ],
        out_specs=pl.BlockSpec((blk, fo), lambda i: (i, 0)),
        out_shape=jax.ShapeDtypeStruct((n, fo), jnp.float32),
    )(x, w, b.reshape(1, fo))


def _gcnpost_body(agg_ref, h_ref, d2_ref, b_ref, o_ref, *, act):
    h = agg_ref[...] + d2_ref[...] * h_ref[...] + b_ref[...]
    o_ref[...] = _apply_act(h, act)


def _gcnpost(agg, h, dis2, b, act):
    n, f = agg.shape
    blk = min(_BLK, n)
    return pl.pallas_call(
        functools.partial(_gcnpost_body, act=act),
        grid=(pl.cdiv(n, blk),),
        in_specs=[
            pl.BlockSpec((blk, f), lambda i: (i, 0)),
            pl.BlockSpec((blk, f), lambda i: (i, 0)),
            pl.BlockSpec((blk, 1), lambda i: (i, 0)),
            pl.BlockSpec((1, f), lambda i: (0, 0)),
        ],
        out_specs=pl.BlockSpec((blk, f), lambda i: (i, 0)),
        out_shape=jax.ShapeDtypeStruct((n, f), jnp.float32),
    )(agg, h, dis2, b.reshape(1, f))


def _trpost_body(a_ref, c_ref, s_ref, b_ref, o_ref, *, act):
    o_ref[...] = _apply_act(a_ref[...] + c_ref[...] + s_ref[...] * b_ref[...],
                            act)


def _trpost(a, c, s, b, act):
    n, f = a.shape
    blk = min(_BLK, n)
    return pl.pallas_call(
        functools.partial(_trpost_body, act=act),
        grid=(pl.cdiv(n, blk),),
        in_specs=[
            pl.BlockSpec((blk, f), lambda i: (i, 0)),
            pl.BlockSpec((blk, f), lambda i: (i, 0)),
            pl.BlockSpec((blk, 1), lambda i: (i, 0)),
            pl.BlockSpec((1, f), lambda i: (0, 0)),
        ],
        out_specs=pl.BlockSpec((blk, f), lambda i: (i, 0)),
        out_shape=jax.ShapeDtypeStruct((n, f), jnp.float32),
    )(a, c, s, b.reshape(1, f))


def _post_body(a_ref, c_ref, o_ref, *, act):
    o_ref[...] = _apply_act(a_ref[...] + c_ref[...], act)


def _post(a, c, act):
    n, f = a.shape
    blk = min(_BLK, n)
    c_rows = c.shape[0]
    c_blk = blk if c_rows == n else 1
    c_map = (lambda i: (i, 0)) if c_rows == n else (lambda i: (0, 0))
    return pl.pallas_call(
        functools.partial(_post_body, act=act),
        grid=(pl.cdiv(n, blk),),
        in_specs=[
            pl.BlockSpec((blk, f), lambda i: (i, 0)),
            pl.BlockSpec((c_blk, f), c_map),
        ],
        out_specs=pl.BlockSpec((blk, f), lambda i: (i, 0)),
        out_shape=jax.ShapeDtypeStruct((n, f), jnp.float32),
    )(a, c)


def _edge_softmax(score, dst, n):
    m = jax.ops.segment_max(score, dst, num_segments=n)
    m = jnp.where(jnp.isfinite(m), m, 0.0)
    e = jnp.exp(score - m[dst])
    s = jax.ops.segment_sum(e, dst, num_segments=n)
    return e / (s[dst] + 1e-16)


def _gcn_layer(x, p, src, dst, norm, dis2, act, n):
    fi, fo = p["W"].shape
    if fi <= fo:
        aggx = jax.ops.segment_sum(x[src] * norm, dst, num_segments=n)
        aggx = aggx + dis2 * x
        return _mm(aggx, p["W"], p["b"], act)
    h = _mm(x, p["W"], jnp.zeros((fo,), jnp.float32))
    agg = jax.ops.segment_sum(h[src] * norm, dst, num_segments=n)
    return _gcnpost(agg, h, dis2, p["b"], act)


def _tr_layer(x, p, src, dst, act, n):
    fi, dh = p["Wq"].shape
    inv = 1.0 / math.sqrt(float(dh))
    if fi <= dh:
        core = p["Wq"] @ p["Wk"].T
        u = p["Wq"] @ p["bk"]
        w = p["Wk"] @ p["bq"]
        c = jnp.dot(p["bq"], p["bk"])
        xM = _mm(x, core, jnp.zeros((fi,), jnp.float32))
        a1 = x @ u
        a2 = x @ w
        score = (jnp.sum(xM[dst] * x[src], axis=1)
                 + a1[dst] + a2[src] + c) * inv
        alpha = _edge_softmax(score, dst, n)
        aggx = jax.ops.segment_sum(x[src] * alpha[:, None], dst,
                                   num_segments=n)
        salpha = jax.ops.segment_sum(alpha, dst, num_segments=n)
        aggv = _mm(aggx, p["Wv"], jnp.zeros((dh,), jnp.float32))
        xs = _mm(x, p["Ws"], p["bs"])
        return _trpost(aggv, xs, salpha[:, None], p["bv"], act)
    q = _mm(x, p["Wq"], p["bq"])
    k = _mm(x, p["Wk"], p["bk"])
    v = _mm(x, p["Wv"], p["bv"])
    xs = _mm(x, p["Ws"], p["bs"])
    score = jnp.sum(q[dst] * k[src], axis=1) * inv
    alpha = _edge_softmax(score, dst, n)
    agg = jax.ops.segment_sum(v[src] * alpha[:, None], dst, num_segments=n)
    return _post(agg, xs, act)


def kernel(x, edge_index, params):
    n = x.shape[0]
    src, dst = edge_index[0], edge_index[1]
    g = params["gcn"]
    t = params["tr"]

    deg = jax.ops.segment_sum(
        jnp.ones(dst.shape, jnp.float32), dst, num_segments=n) + 1.0
    dis = jax.lax.rsqrt(deg)
    norm = (dis[src] * dis[dst])[:, None]
    dis2 = (dis * dis)[:, None]

    x = _gcn_layer(x, g[0], src, dst, norm, dis2, "relu", n)
    x = _gcn_layer(x, g[1], src, dst, norm, dis2, "relu", n)
    x = _gcn_layer(x, g[2], src, dst, norm, dis2, "softmax", n)
    x = _tr_layer(x, t[0], src, dst, "relu", n)
    x = _tr_layer(x, t[1], src, dst, "relu", n)
    x = _tr_layer(x, t[2], src, dst, "relu_softmax", n)
    x = _gcn_layer(x, g[3], src, dst, norm, dis2, "relu", n)
    x = _gcn_layer(x, g[4], src, dst, norm, dis2, "relu", n)
    x = _gcn_layer(x, g[5], src, dst, norm, dis2, "softmax", n)
    x = _post(x, jnp.zeros((1, x.shape[1]), jnp.float32), "softmax5concat")
    x5 = x[:, :5]
    x = _gcn_layer(x, g[6], src, dst, norm, dis2, "relu", n)
    x = _gcn_layer(x, g[7], src, dst, norm, dis2, "relu", n)
    x = _gcn_layer(x, g[8], src, dst, norm, dis2, "softmax", n)
    return (x5, x)


# speedup vs baseline: 1.1672x; 1.1672x over previous
import functools
import math

import jax
import jax.numpy as jnp
from jax.experimental import pallas as pl

_BLK = 1000


def _apply_act(h, act):
    if act == "relu":
        return jnp.maximum(h, 0.0)
    if act in ("softmax", "relu_softmax"):
        if act == "relu_softmax":
            h = jnp.maximum(h, 0.0)
        m = jnp.max(h, axis=1, keepdims=True)
        e = jnp.exp(h - m)
        return e / jnp.sum(e, axis=1, keepdims=True)
    if act == "softmax5concat":
        col = jax.lax.broadcasted_iota(jnp.int32, h.shape, 1)
        in5 = col < 5
        z = jnp.where(in5, h, -jnp.inf)
        m = jnp.max(z, axis=1, keepdims=True)
        e = jnp.where(in5, jnp.exp(z - m), 0.0)
        sm = e / jnp.sum(e, axis=1, keepdims=True)
        return jnp.where(in5, sm, h)
    return h


def _mm_body(x_ref, w_ref, b_ref, o_ref, *, act):
    h = jnp.dot(x_ref[...], w_ref[...],
                preferred_element_type=jnp.float32,
                precision=jax.lax.Precision.HIGHEST)
    o_ref[...] = _apply_act(h + b_ref[...], act)


def _mm(x, w, b, act="none"):
    n, fi = x.shape
    fo = w.shape[1]
    blk = min(_BLK, n)
    return pl.pallas_call(
        functools.partial(_mm_body, act=act),
        grid=(pl.cdiv(n, blk),),
        in_specs=[
            pl.BlockSpec((blk, fi), lambda i: (i, 0)),
            pl.BlockSpec((fi, fo), lambda i: (0, 0)),
            pl.BlockSpec((1, fo), lambda i: (0, 0)),
        ],
        out_specs=pl.BlockSpec((blk, fo), lambda i: (i, 0)),
        out_shape=jax.ShapeDtypeStruct((n, fo), jnp.float32),
    )(x, w, b.reshape(1, fo))


def _gcnpost_body(agg_ref, h_ref, d2_ref, b_ref, o_ref, *, act):
    h = agg_ref[...] + d2_ref[...] * h_ref[...] + b_ref[...]
    o_ref[...] = _apply_act(h, act)


def _gcnpost(agg, h, dis2, b, act):
    n, f = agg.shape
    blk = min(_BLK, n)
    return pl.pallas_call(
        functools.partial(_gcnpost_body, act=act),
        grid=(pl.cdiv(n, blk),),
        in_specs=[
            pl.BlockSpec((blk, f), lambda i: (i, 0)),
            pl.BlockSpec((blk, f), lambda i: (i, 0)),
            pl.BlockSpec((blk, 1), lambda i: (i, 0)),
            pl.BlockSpec((1, f), lambda i: (0, 0)),
        ],
        out_specs=pl.BlockSpec((blk, f), lambda i: (i, 0)),
        out_shape=jax.ShapeDtypeStruct((n, f), jnp.float32),
    )(agg, h, dis2, b.reshape(1, f))


def _trpost_body(a_ref, c_ref, s_ref, b_ref, o_ref, *, act):
    o_ref[...] = _apply_act(a_ref[...] + c_ref[...] + s_ref[...] * b_ref[...],
                            act)


def _trpost(a, c, s, b, act):
    n, f = a.shape
    blk = min(_BLK, n)
    return pl.pallas_call(
        functools.partial(_trpost_body, act=act),
        grid=(pl.cdiv(n, blk),),
        in_specs=[
            pl.BlockSpec((blk, f), lambda i: (i, 0)),
            pl.BlockSpec((blk, f), lambda i: (i, 0)),
            pl.BlockSpec((blk, 1), lambda i: (i, 0)),
            pl.BlockSpec((1, f), lambda i: (0, 0)),
        ],
        out_specs=pl.BlockSpec((blk, f), lambda i: (i, 0)),
        out_shape=jax.ShapeDtypeStruct((n, f), jnp.float32),
    )(a, c, s, b.reshape(1, f))


def _post_body(a_ref, c_ref, o_ref, *, act):
    o_ref[...] = _apply_act(a_ref[...] + c_ref[...], act)


def _post(a, c, act):
    n, f = a.shape
    blk = min(_BLK, n)
    c_rows = c.shape[0]
    c_blk = blk if c_rows == n else 1
    c_map = (lambda i: (i, 0)) if c_rows == n else (lambda i: (0, 0))
    return pl.pallas_call(
        functools.partial(_post_body, act=act),
        grid=(pl.cdiv(n, blk),),
        in_specs=[
            pl.BlockSpec((blk, f), lambda i: (i, 0)),
            pl.BlockSpec((c_blk, f), c_map),
        ],
        out_specs=pl.BlockSpec((blk, f), lambda i: (i, 0)),
        out_shape=jax.ShapeDtypeStruct((n, f), jnp.float32),
    )(a, c)


def _edge_softmax(score, dst, n):
    m = jax.ops.segment_max(score, dst, num_segments=n)
    m = jnp.where(jnp.isfinite(m), m, 0.0)
    e = jnp.exp(score - m[dst])
    s = jax.ops.segment_sum(e, dst, num_segments=n)
    return e / (s[dst] + 1e-16)


def _gcn_layer(x, p, src, dst, norm, dis2, act, n):
    fi, fo = p["W"].shape
    if fi <= fo:
        aggx = jax.ops.segment_sum(x[src] * norm, dst, num_segments=n)
        aggx = aggx + dis2 * x
        return _mm(aggx, p["W"], p["b"], act)
    h = _mm(x, p["W"], jnp.zeros((fo,), jnp.float32))
    agg = jax.ops.segment_sum(h[src] * norm, dst, num_segments=n)
    return _gcnpost(agg, h, dis2, p["b"], act)


def _tr_layer(x, p, src, dst, act, n):
    dh = p["Wq"].shape[1]
    inv = 1.0 / math.sqrt(float(dh))
    q = _mm(x, p["Wq"], p["bq"])
    k = _mm(x, p["Wk"], p["bk"])
    v = _mm(x, p["Wv"], p["bv"])
    xs = _mm(x, p["Ws"], p["bs"])
    score = jnp.sum(q[dst] * k[src], axis=1) * inv
    alpha = _edge_softmax(score, dst, n)
    agg = jax.ops.segment_sum(v[src] * alpha[:, None], dst, num_segments=n)
    return _post(agg, xs, act)


def kernel(x, edge_index, params):
    n = x.shape[0]
    src, dst = edge_index[0], edge_index[1]
    g = params["gcn"]
    t = params["tr"]

    deg = jax.ops.segment_sum(
        jnp.ones(dst.shape, jnp.float32), dst, num_segments=n) + 1.0
    dis = jax.lax.rsqrt(deg)
    norm = (dis[src] * dis[dst])[:, None]
    dis2 = (dis * dis)[:, None]

    x = _gcn_layer(x, g[0], src, dst, norm, dis2, "relu", n)
    x = _gcn_layer(x, g[1], src, dst, norm, dis2, "relu", n)
    x = _gcn_layer(x, g[2], src, dst, norm, dis2, "softmax", n)
    x = _tr_layer(x, t[0], src, dst, "relu", n)
    x = _tr_layer(x, t[1], src, dst, "relu", n)
    x = _tr_layer(x, t[2], src, dst, "relu_softmax", n)
    x = _gcn_layer(x, g[3], src, dst, norm, dis2, "relu", n)
    x = _gcn_layer(x, g[4], src, dst, norm, dis2, "relu", n)
    x = _gcn_layer(x, g[5], src, dst, norm, dis2, "softmax", n)
    x = _post(x, jnp.zeros((1, x.shape[1]), jnp.float32), "softmax5concat")
    x5 = x[:, :5]
    x = _gcn_layer(x, g[6], src, dst, norm, dis2, "relu", n)
    x = _gcn_layer(x, g[7], src, dst, norm, dis2, "relu", n)
    x = _gcn_layer(x, g[8], src, dst, norm, dis2, "softmax", n)
    return (x5, x)
